# baseline (speedup 1.0000x reference)
import functools

import jax
from jax import lax
import jax.numpy as jnp
from jax.experimental import pallas as pl
from jax.experimental.pallas import tpu as pltpu
from jax.experimental.pallas import tpu_sc as plsc

_TAU = 0.6
_EPS = 1e-06

_NC = 2
_NS = 16
_NW = _NC * _NS
_L = 16


def _phase1_kernel(x_ref, conf_ref, idx_ref, hist_ref, tc_ref):
    b = pl.program_id(0)
    h = pl.program_id(1)

    @pl.when(jnp.logical_and(b == 0, h == 0))
    def _init():
        hist_ref[...] = jnp.zeros_like(hist_ref)

    x = x_ref[0]
    C = x.shape[0]
    m = x[0]
    idx = jnp.zeros(m.shape, jnp.int32)
    for c in range(1, C):
        xc = x[c]
        gt = xc > m
        m = jnp.where(gt, xc, m)
        idx = jnp.where(gt, c, idx)
    s = jnp.exp(x[0] - m)
    for c in range(1, C):
        s = s + jnp.exp(x[c] - m)
    conf = 1.0 / s
    conf_ref[0] = conf
    idx_ref[0] = idx

    high = (conf > _TAU).astype(jnp.float32)
    cls = jax.lax.broadcasted_iota(jnp.int32, (C,) + idx.shape, 0)
    onehot = jnp.where(idx[None] == cls, high[None], 0.0)
    hist_ref[...] += jnp.sum(onehot, axis=(1, 2)).reshape(1, C)

    nb = pl.num_programs(0)
    nh = pl.num_programs(1)

    @pl.when(jnp.logical_and(b == nb - 1, h == nh - 1))
    def _finalize():
        sigma = hist_ref[...]
        sigma_hat = sigma / jnp.maximum(jnp.max(sigma), _EPS)
        t_c = sigma_hat / (2.0 - jnp.minimum(sigma_hat, 1.0)) * _TAU
        pad = tc_ref.shape[1] - C
        tc_ref[...] = jnp.pad(t_c, ((0, 0), (0, pad)))


def _make_phase2_sc(n_pixels):
    per_w = n_pixels // _NW
    CH = 16384
    n_chunks = per_w // CH
    mesh = plsc.VectorSubcoreMesh(
        core_axis_name="c", subcore_axis_name="s",
        num_cores=_NC, num_subcores=_NS,
    )

    @functools.partial(
        pl.kernel,
        out_type=jax.ShapeDtypeStruct((n_pixels,), jnp.int32),
        mesh=mesh,
        compiler_params=pltpu.CompilerParams(needs_layout_passes=False),
        scratch_types=[
            pltpu.VMEM((32,), jnp.float32),
            pltpu.VMEM((CH,), jnp.float32),
            pltpu.VMEM((CH,), jnp.int32),
            pltpu.VMEM((CH,), jnp.int32),
        ],
    )
    def phase2(tc_hbm, conf_hbm, idx_hbm, delta_hbm,
               tc_v, conf_v, idx_v, out_v):
        wid = lax.axis_index("s") * _NC + lax.axis_index("c")
        pltpu.sync_copy(tc_hbm, tc_v)

        base = wid * per_w
        for k in range(n_chunks):
            off = base + k * CH
            pltpu.sync_copy(conf_hbm.at[pl.ds(off, CH)], conf_v)
            pltpu.sync_copy(idx_hbm.at[pl.ds(off, CH)], idx_v)

            def body(i, carry):
                iv = idx_v[pl.ds(i * _L, _L)]
                cv = conf_v[pl.ds(i * _L, _L)]
                t = plsc.load_gather(tc_v, [iv])
                out_v[pl.ds(i * _L, _L)] = (cv > t).astype(jnp.int32)
                return carry

            lax.fori_loop(0, CH // _L, body, 0)
            pltpu.sync_copy(out_v, delta_hbm.at[pl.ds(off, CH)])

    return phase2


@jax.jit
def kernel(logits):
    B, C, H, W = logits.shape
    BH = 64

    conf, idx, _sigma, tc32 = pl.pallas_call(
        _phase1_kernel,
        grid=(B, H // BH),
        in_specs=[
            pl.BlockSpec((1, C, BH, W), lambda b, h: (b, 0, h, 0)),
        ],
        out_specs=[
            pl.BlockSpec((1, BH, W), lambda b, h: (b, h, 0)),
            pl.BlockSpec((1, BH, W), lambda b, h: (b, h, 0)),
            pl.BlockSpec((1, C), lambda b, h: (0, 0)),
            pl.BlockSpec((1, 32), lambda b, h: (0, 0)),
        ],
        out_shape=[
            jax.ShapeDtypeStruct((B, H, W), jnp.float32),
            jax.ShapeDtypeStruct((B, H, W), jnp.int32),
            jax.ShapeDtypeStruct((1, C), jnp.float32),
            jax.ShapeDtypeStruct((1, 32), jnp.float32),
        ],
    )(logits)

    n_pixels = B * H * W

    delta_i = _make_phase2_sc(n_pixels)(
        tc32.reshape(32), conf.reshape(n_pixels), idx.reshape(n_pixels)
    )

    delta = delta_i.reshape(B, H, W).astype(jnp.bool_)
    return delta, tc32.reshape(32)[:C]


# speedup vs baseline: 1.1552x; 1.1552x over previous
import functools

import jax
from jax import lax
import jax.numpy as jnp
from jax.experimental import pallas as pl
from jax.experimental.pallas import tpu as pltpu
from jax.experimental.pallas import tpu_sc as plsc

_TAU = 0.6
_EPS = 1e-06

_NC = 2
_NS = 16
_NW = _NC * _NS
_L = 16


def _phase1_kernel(x_ref, conf_ref, idx_ref, hist_ref, tc_ref):
    b = pl.program_id(0)
    h = pl.program_id(1)

    @pl.when(jnp.logical_and(b == 0, h == 0))
    def _init():
        hist_ref[...] = jnp.zeros_like(hist_ref)

    x = x_ref[0]
    C = x.shape[0]
    m = x[0]
    idx = jnp.zeros(m.shape, jnp.int32)
    for c in range(1, C):
        xc = x[c]
        gt = xc > m
        m = jnp.where(gt, xc, m)
        idx = jnp.where(gt, c, idx)
    s = jnp.exp(x[0] - m)
    for c in range(1, C):
        s = s + jnp.exp(x[c] - m)
    conf = 1.0 / s
    conf_ref[...] = conf
    idx_ref[...] = idx

    high = (conf > _TAU).astype(jnp.float32)
    cls = jax.lax.broadcasted_iota(jnp.int32, (C,) + idx.shape, 0)
    onehot = jnp.where(idx[None] == cls, high[None], 0.0)
    hist_ref[...] += jnp.sum(onehot, axis=(1, 2)).reshape(1, C)

    nb = pl.num_programs(0)
    nh = pl.num_programs(1)

    @pl.when(jnp.logical_and(b == nb - 1, h == nh - 1))
    def _finalize():
        sigma = hist_ref[...]
        sigma_hat = sigma / jnp.maximum(jnp.max(sigma), _EPS)
        t_c = sigma_hat / (2.0 - jnp.minimum(sigma_hat, 1.0)) * _TAU
        pad = tc_ref.shape[1] - C
        tc_ref[...] = jnp.pad(t_c, ((0, 0), (0, pad)))


def _make_phase2_sc(n_rows, n_cols):
    rows_w = n_rows // _NW
    CR = 32
    n_chunks = rows_w // CR
    mesh = plsc.VectorSubcoreMesh(
        core_axis_name="c", subcore_axis_name="s",
        num_cores=_NC, num_subcores=_NS,
    )

    @functools.partial(
        pl.kernel,
        out_type=jax.ShapeDtypeStruct((n_rows, n_cols), jnp.int32),
        mesh=mesh,
        compiler_params=pltpu.CompilerParams(needs_layout_passes=False),
        scratch_types=[
            pltpu.VMEM((32,), jnp.float32),
            pltpu.VMEM((CR, n_cols), jnp.float32),
            pltpu.VMEM((CR, n_cols), jnp.int32),
            pltpu.VMEM((CR, n_cols), jnp.int32),
        ],
    )
    def phase2(tc_hbm, conf_hbm, idx_hbm, delta_hbm,
               tc_v, conf_v, idx_v, out_v):
        wid = lax.axis_index("s") * _NC + lax.axis_index("c")
        pltpu.sync_copy(tc_hbm, tc_v)

        base = wid * rows_w
        for k in range(n_chunks):
            row0 = base + k * CR
            pltpu.sync_copy(conf_hbm.at[pl.ds(row0, CR)], conf_v)
            pltpu.sync_copy(idx_hbm.at[pl.ds(row0, CR)], idx_v)

            def body(i, carry):
                r = i // (n_cols // _L)
                c = (i % (n_cols // _L)) * _L
                iv = idx_v[r, pl.ds(c, _L)]
                cv = conf_v[r, pl.ds(c, _L)]
                t = plsc.load_gather(tc_v, [iv])
                out_v[r, pl.ds(c, _L)] = (cv > t).astype(jnp.int32)
                return carry

            lax.fori_loop(0, CR * (n_cols // _L), body, 0)
            pltpu.sync_copy(out_v, delta_hbm.at[pl.ds(row0, CR)])

    return phase2


@jax.jit
def kernel(logits):
    B, C, H, W = logits.shape
    BH = 64
    R = B * H
    nh = H // BH

    conf, idx, _sigma, tc32 = pl.pallas_call(
        _phase1_kernel,
        grid=(B, H // BH),
        in_specs=[
            pl.BlockSpec((1, C, BH, W), lambda b, h: (b, 0, h, 0)),
        ],
        out_specs=[
            pl.BlockSpec((BH, W), lambda b, h: (b * nh + h, 0)),
            pl.BlockSpec((BH, W), lambda b, h: (b * nh + h, 0)),
            pl.BlockSpec((1, C), lambda b, h: (0, 0)),
            pl.BlockSpec((1, 32), lambda b, h: (0, 0)),
        ],
        out_shape=[
            jax.ShapeDtypeStruct((R, W), jnp.float32),
            jax.ShapeDtypeStruct((R, W), jnp.int32),
            jax.ShapeDtypeStruct((1, C), jnp.float32),
            jax.ShapeDtypeStruct((1, 32), jnp.float32),
        ],
    )(logits)

    delta_i = _make_phase2_sc(R, W)(tc32.reshape(32), conf, idx)

    delta = (delta_i != 0).reshape(B, H, W)
    return delta, tc32.reshape(32)[:C]


# speedup vs baseline: 1.1990x; 1.0379x over previous
import functools

import jax
from jax import lax
import jax.numpy as jnp
from jax.experimental import pallas as pl
from jax.experimental.pallas import tpu as pltpu
from jax.experimental.pallas import tpu_sc as plsc

_TAU = 0.6
_EPS = 1e-06

_NC = 2
_NS = 16
_NW = _NC * _NS
_L = 16


def _phase1_kernel(x_ref, conf_ref, idxp_ref, hist_ref, tc_ref):
    b = pl.program_id(0)
    h = pl.program_id(1)

    @pl.when(jnp.logical_and(b == 0, h == 0))
    def _init():
        hist_ref[...] = jnp.zeros_like(hist_ref)

    x = x_ref[0]
    C = x.shape[0]
    m = x[0]
    idx = jnp.zeros(m.shape, jnp.int32)
    for c in range(1, C):
        xc = x[c]
        gt = xc > m
        m = jnp.where(gt, xc, m)
        idx = jnp.where(gt, c, idx)
    s = jnp.exp(x[0] - m)
    for c in range(1, C):
        s = s + jnp.exp(x[c] - m)
    conf = 1.0 / s
    conf_ref[...] = conf

    idxp_ref[...] = (idx[:, 0:128]
                     | (idx[:, 128:256] << 8)
                     | (idx[:, 256:384] << 16)
                     | (idx[:, 384:512] << 24))

    high = (conf > _TAU).astype(jnp.float32)
    cls = jax.lax.broadcasted_iota(jnp.int32, (C,) + idx.shape, 0)
    onehot = jnp.where(idx[None] == cls, high[None], 0.0)
    hist_ref[...] += jnp.sum(onehot, axis=(1, 2)).reshape(1, C)

    nb = pl.num_programs(0)
    nh = pl.num_programs(1)

    @pl.when(jnp.logical_and(b == nb - 1, h == nh - 1))
    def _finalize():
        sigma = hist_ref[...]
        sigma_hat = sigma / jnp.maximum(jnp.max(sigma), _EPS)
        t_c = sigma_hat / (2.0 - jnp.minimum(sigma_hat, 1.0)) * _TAU
        pad = tc_ref.shape[1] - C
        tc_ref[...] = jnp.pad(t_c, ((0, 0), (0, pad)))


def _make_phase2_sc(n_rows, n_cols):
    rows_w = n_rows // _NW
    CR = 32
    n_chunks = rows_w // CR
    n_words = n_cols // 4
    n_q = n_words // _L
    mesh = plsc.VectorSubcoreMesh(
        core_axis_name="c", subcore_axis_name="s",
        num_cores=_NC, num_subcores=_NS,
    )

    @functools.partial(
        pl.kernel,
        out_type=jax.ShapeDtypeStruct((n_rows, n_words), jnp.int32),
        mesh=mesh,
        compiler_params=pltpu.CompilerParams(needs_layout_passes=False),
        scratch_types=[
            pltpu.VMEM((32,), jnp.float32),
            pltpu.VMEM((CR, n_cols), jnp.float32),
            pltpu.VMEM((CR, n_words), jnp.int32),
            pltpu.VMEM((CR, n_words), jnp.int32),
        ],
    )
    def phase2(tc_hbm, conf_hbm, idxp_hbm, delta_hbm,
               tc_v, conf_v, idxp_v, out_v):
        wid = lax.axis_index("s") * _NC + lax.axis_index("c")
        pltpu.sync_copy(tc_hbm, tc_v)

        base = wid * rows_w
        for k in range(n_chunks):
            row0 = base + k * CR
            pltpu.sync_copy(conf_hbm.at[pl.ds(row0, CR)], conf_v)
            pltpu.sync_copy(idxp_hbm.at[pl.ds(row0, CR)], idxp_v)

            def body(i, carry):
                r = i // n_q
                q = (i % n_q) * _L
                pv = idxp_v[r, pl.ds(q, _L)]
                out = jnp.zeros((_L,), jnp.int32)
                for o in range(4):
                    iv = (pv >> (8 * o)) & 0xFF
                    cv = conf_v[r, pl.ds(o * 128 + q, _L)]
                    t = plsc.load_gather(tc_v, [iv])
                    out = out | jnp.where(cv > t, 1 << (8 * o), 0)
                out_v[r, pl.ds(q, _L)] = out
                return carry

            lax.fori_loop(0, CR * n_q, body, 0)
            pltpu.sync_copy(out_v, delta_hbm.at[pl.ds(row0, CR)])

    return phase2


@jax.jit
def kernel(logits):
    B, C, H, W = logits.shape
    BH = 64
    R = B * H
    nh = H // BH

    conf, idxp, _sigma, tc32 = pl.pallas_call(
        _phase1_kernel,
        grid=(B, H // BH),
        in_specs=[
            pl.BlockSpec((1, C, BH, W), lambda b, h: (b, 0, h, 0)),
        ],
        out_specs=[
            pl.BlockSpec((BH, W), lambda b, h: (b * nh + h, 0)),
            pl.BlockSpec((BH, W // 4), lambda b, h: (b * nh + h, 0)),
            pl.BlockSpec((1, C), lambda b, h: (0, 0)),
            pl.BlockSpec((1, 32), lambda b, h: (0, 0)),
        ],
        out_shape=[
            jax.ShapeDtypeStruct((R, W), jnp.float32),
            jax.ShapeDtypeStruct((R, W // 4), jnp.int32),
            jax.ShapeDtypeStruct((1, C), jnp.float32),
            jax.ShapeDtypeStruct((1, 32), jnp.float32),
        ],
    )(logits)

    deltap = _make_phase2_sc(R, W)(tc32.reshape(32), conf, idxp)

    d8 = lax.bitcast_convert_type(deltap, jnp.int8)
    delta = (d8 != 0).transpose(0, 2, 1).reshape(B, H, W)
    return delta, tc32.reshape(32)[:C]
